# baseline (speedup 1.0000x reference)
import math

import jax
import jax.numpy as jnp
from jax.experimental import pallas as pl
from jax.experimental.pallas import tpu as pltpu

_B, _H, _W, _A, _C = 32, 56, 56, 9, 6
_N = 20
_NB = 16
_ANCHORS = [
    (s * math.sqrt(r) / 224.0, s / math.sqrt(r) / 224.0)
    for s in (32, 64, 128)
    for r in (0.5, 1.0, 2.0)
]


def _loss_kernel(idx_smem, pred_ref, bbc_ref, ixc_ref, out_ref, scr):
    g = pl.program_id(0)
    f32 = jnp.float32

    x = pred_ref[:, :, 4, :, :]
    p_all = jax.nn.sigmoid(x)
    s0 = jnp.sum(-jnp.maximum(jnp.log(1.0 - p_all), -100.0))

    npos_b = jnp.float32(0.0)
    sobj_b = jnp.float32(0.0)
    sncorr_b = jnp.float32(0.0)
    scoord_b = jnp.float32(0.0)
    n_c = jax.lax.broadcasted_iota(jnp.int32, (_N, 1), 0)
    n_r = jax.lax.broadcasted_iota(jnp.int32, (1, _N), 1)
    lane = jax.lax.broadcasted_iota(jnp.int32, (_N, _W), 1)

    for i in range(_NB):
        b = g * _NB + i
        bbc = bbc_ref[b]
        cx, cy = bbc[:, 0:1], bbc[:, 1:2]
        w, h = bbc[:, 2:3], bbc[:, 3:4]
        pk = ixc_ref[b]
        valid_c = (pk & 1) != 0
        best_c = (pk >> 1) & 15
        gx_c = (pk >> 5) & 63
        gy_c = (pk >> 11) & 63
        tx = cx * _W - gx_c.astype(f32)
        ty = cy * _H - gy_c.astype(f32)
        baw = jnp.zeros(best_c.shape, f32)
        bah = jnp.zeros(best_c.shape, f32)
        for k, (awk, ahk) in enumerate(_ANCHORS):
            baw = jnp.where(best_c == k, awk, baw)
            bah = jnp.where(best_c == k, ahk, bah)
        tw = jnp.log(w / baw + 1e-16)
        th = jnp.log(h / bah + 1e-16)
        flat_c = (gy_c * _W + gx_c) * _A + best_c

        id_c = jnp.where(valid_c, flat_c, -1 - n_c)
        id_r = jnp.transpose(id_c, (1, 0))
        killed = jnp.any((id_c == id_r) & (n_r > n_c), axis=1, keepdims=True)
        alive = (valid_c & ~killed).astype(f32)

        for n in range(_N):
            s = idx_smem[b, n, 0]
            bestn = (s >> 1) & 15
            gyn = (s >> 11) & 63
            scr[n] = pred_ref[i, bestn, 0:5, gyn, :]
        sel = lane == gx_c
        v = [
            jnp.sum(jnp.where(sel, scr[:, c, :], 0.0), axis=1, keepdims=True)
            for c in range(5)
        ]
        coord_n = (
            (v[0] - tx) ** 2 + (v[1] - ty) ** 2 + (v[2] - tw) ** 2 + (v[3] - th) ** 2
        )
        pv = jax.nn.sigmoid(v[4])
        obj_n = -jnp.maximum(jnp.log(pv), -100.0)
        ncorr_n = -jnp.maximum(jnp.log(1.0 - pv), -100.0)

        npos_b += jnp.sum(alive)
        sobj_b += jnp.sum(alive * obj_n)
        sncorr_b += jnp.sum(alive * ncorr_n)
        scoord_b += jnp.sum(alive * coord_n)

    lane_o = jax.lax.broadcasted_iota(jnp.int32, (1, 128), 1)
    delta = (
        jnp.where(lane_o == 0, s0, 0.0)
        + jnp.where(lane_o == 1, npos_b, 0.0)
        + jnp.where(lane_o == 2, sobj_b, 0.0)
        + jnp.where(lane_o == 3, sncorr_b, 0.0)
        + jnp.where(lane_o == 4, scoord_b, 0.0)
    )

    @pl.when(g == 0)
    def _zero():
        out_ref[...] = jnp.zeros_like(out_ref)

    out_ref[...] = out_ref[...] + delta

    @pl.when(g == _B // _NB - 1)
    def _finalize():
        a = out_ref[...]
        s0s = jnp.sum(jnp.where(lane_o == 0, a, 0.0))
        npos = jnp.sum(jnp.where(lane_o == 1, a, 0.0))
        sobj = jnp.sum(jnp.where(lane_o == 2, a, 0.0))
        sncorr = jnp.sum(jnp.where(lane_o == 3, a, 0.0))
        scoord = jnp.sum(jnp.where(lane_o == 4, a, 0.0))
        n_neg = jnp.float32(_B * _H * _W * _A) - npos
        coord_loss = 5.0 * scoord
        noobj_loss = 0.5 * (s0s - sncorr)
        coord_loss = jnp.where(npos > 0, coord_loss / npos, coord_loss)
        obj_loss = jnp.where(npos > 0, sobj / npos, sobj)
        noobj_loss = jnp.where(n_neg > 0, noobj_loss / n_neg, noobj_loss)
        total_loss = coord_loss + obj_loss + noobj_loss
        out_ref[...] = (
            jnp.where(lane_o == 0, total_loss, 0.0)
            + jnp.where(lane_o == 1, coord_loss, 0.0)
            + jnp.where(lane_o == 2, obj_loss, 0.0)
            + jnp.where(lane_o == 3, noobj_loss, 0.0)
        )


def kernel(predictions, bboxes):
    B, H, W, A, C = predictions.shape
    pred_t = jnp.transpose(predictions, (0, 3, 4, 1, 2))

    anchors = jnp.asarray(_ANCHORS, dtype=jnp.float32)
    cx, cy = bboxes[..., 0], bboxes[..., 1]
    w, h = bboxes[..., 2], bboxes[..., 3]
    valid = ~jnp.all(bboxes == 0.0, axis=-1)
    gx = jnp.clip(jnp.floor(cx * W).astype(jnp.int32), 0, W - 1)
    gy = jnp.clip(jnp.floor(cy * H).astype(jnp.int32), 0, H - 1)
    aw = anchors[:, 0][None, None, :]
    ah = anchors[:, 1][None, None, :]
    inter = jnp.minimum(w[..., None], aw) * jnp.minimum(h[..., None], ah)
    union = (w * h)[..., None] + aw * ah - inter
    iou = inter / (union + 1e-16)
    best = jnp.argmax(iou, axis=-1).astype(jnp.int32)

    pk = (((gy * 64 + gx) * 16 + best) * 2 + valid.astype(jnp.int32)).reshape(
        B, _N, 1
    )

    grid_spec = pltpu.PrefetchScalarGridSpec(
        num_scalar_prefetch=1,
        grid=(B // _NB,),
        in_specs=[
            pl.BlockSpec((_NB, A, 5, H, W), lambda b, s: (b, 0, 0, 0, 0)),
            pl.BlockSpec((B, _N, 4), lambda b, s: (0, 0, 0)),
            pl.BlockSpec((B, _N, 1), lambda b, s: (0, 0, 0)),
        ],
        out_specs=pl.BlockSpec((1, 128), lambda b, s: (0, 0)),
        scratch_shapes=[pltpu.VMEM((_N, 5, _W), jnp.float32)],
    )
    acc = pl.pallas_call(
        _loss_kernel,
        grid_spec=grid_spec,
        out_shape=jax.ShapeDtypeStruct((1, 128), jnp.float32),
    )(pk, pred_t, bboxes, pk)

    cls_loss = jnp.asarray(0.0, jnp.float32)
    return (acc[0, 0], acc[0, 1], acc[0, 2], acc[0, 3], cls_loss)


# speedup vs baseline: 1.0548x; 1.0548x over previous
import math

import jax
import jax.numpy as jnp
from jax.experimental import pallas as pl
from jax.experimental.pallas import tpu as pltpu

_B, _H, _W, _A, _C = 32, 56, 56, 9, 6
_N = 20
_NB = 4
_ANCHORS = [
    (s * math.sqrt(r) / 224.0, s / math.sqrt(r) / 224.0)
    for s in (32, 64, 128)
    for r in (0.5, 1.0, 2.0)
]


def _loss_kernel(idx_smem, pred_ref, bbc_ref, ixc_ref, out_ref, scr):
    g = pl.program_id(0)
    f32 = jnp.float32

    x = pred_ref[:, :, 4, :, :]
    p_all = jax.nn.sigmoid(x)
    s0 = jnp.sum(-jnp.maximum(jnp.log(1.0 - p_all), -100.0))

    npos_b = jnp.float32(0.0)
    sobj_b = jnp.float32(0.0)
    sncorr_b = jnp.float32(0.0)
    scoord_b = jnp.float32(0.0)
    n_c = jax.lax.broadcasted_iota(jnp.int32, (_N, 1), 0)
    n_r = jax.lax.broadcasted_iota(jnp.int32, (1, _N), 1)
    lane = jax.lax.broadcasted_iota(jnp.int32, (_N, _W), 1)

    for i in range(_NB):
        b = g * _NB + i
        bbc = bbc_ref[b]
        cx, cy = bbc[:, 0:1], bbc[:, 1:2]
        w, h = bbc[:, 2:3], bbc[:, 3:4]
        pk = ixc_ref[b]
        valid_c = (pk & 1) != 0
        best_c = (pk >> 1) & 15
        gx_c = (pk >> 5) & 63
        gy_c = (pk >> 11) & 63
        tx = cx * _W - gx_c.astype(f32)
        ty = cy * _H - gy_c.astype(f32)
        baw = jnp.zeros(best_c.shape, f32)
        bah = jnp.zeros(best_c.shape, f32)
        for k, (awk, ahk) in enumerate(_ANCHORS):
            baw = jnp.where(best_c == k, awk, baw)
            bah = jnp.where(best_c == k, ahk, bah)
        tw = jnp.log(w / baw + 1e-16)
        th = jnp.log(h / bah + 1e-16)
        flat_c = (gy_c * _W + gx_c) * _A + best_c

        id_c = jnp.where(valid_c, flat_c, -1 - n_c)
        id_r = jnp.transpose(id_c, (1, 0))
        killed = jnp.any((id_c == id_r) & (n_r > n_c), axis=1, keepdims=True)
        alive = (valid_c & ~killed).astype(f32)

        for n in range(_N):
            s = idx_smem[b, n, 0]
            bestn = (s >> 1) & 15
            gyn = (s >> 11) & 63
            scr[n] = pred_ref[i, bestn, 0:5, gyn, :]
        sel = lane == gx_c
        v = [
            jnp.sum(jnp.where(sel, scr[:, c, :], 0.0), axis=1, keepdims=True)
            for c in range(5)
        ]
        coord_n = (
            (v[0] - tx) ** 2 + (v[1] - ty) ** 2 + (v[2] - tw) ** 2 + (v[3] - th) ** 2
        )
        pv = jax.nn.sigmoid(v[4])
        obj_n = -jnp.maximum(jnp.log(pv), -100.0)
        ncorr_n = -jnp.maximum(jnp.log(1.0 - pv), -100.0)

        npos_b += jnp.sum(alive)
        sobj_b += jnp.sum(alive * obj_n)
        sncorr_b += jnp.sum(alive * ncorr_n)
        scoord_b += jnp.sum(alive * coord_n)

    lane_o = jax.lax.broadcasted_iota(jnp.int32, (1, 128), 1)
    delta = (
        jnp.where(lane_o == 0, s0, 0.0)
        + jnp.where(lane_o == 1, npos_b, 0.0)
        + jnp.where(lane_o == 2, sobj_b, 0.0)
        + jnp.where(lane_o == 3, sncorr_b, 0.0)
        + jnp.where(lane_o == 4, scoord_b, 0.0)
    )

    @pl.when(g == 0)
    def _zero():
        out_ref[...] = jnp.zeros_like(out_ref)

    out_ref[...] = out_ref[...] + delta

    @pl.when(g == _B // _NB - 1)
    def _finalize():
        a = out_ref[...]
        s0s = jnp.sum(jnp.where(lane_o == 0, a, 0.0))
        npos = jnp.sum(jnp.where(lane_o == 1, a, 0.0))
        sobj = jnp.sum(jnp.where(lane_o == 2, a, 0.0))
        sncorr = jnp.sum(jnp.where(lane_o == 3, a, 0.0))
        scoord = jnp.sum(jnp.where(lane_o == 4, a, 0.0))
        n_neg = jnp.float32(_B * _H * _W * _A) - npos
        coord_loss = 5.0 * scoord
        noobj_loss = 0.5 * (s0s - sncorr)
        coord_loss = jnp.where(npos > 0, coord_loss / npos, coord_loss)
        obj_loss = jnp.where(npos > 0, sobj / npos, sobj)
        noobj_loss = jnp.where(n_neg > 0, noobj_loss / n_neg, noobj_loss)
        total_loss = coord_loss + obj_loss + noobj_loss
        out_ref[...] = (
            jnp.where(lane_o == 0, total_loss, 0.0)
            + jnp.where(lane_o == 1, coord_loss, 0.0)
            + jnp.where(lane_o == 2, obj_loss, 0.0)
            + jnp.where(lane_o == 3, noobj_loss, 0.0)
        )


def kernel(predictions, bboxes):
    B, H, W, A, C = predictions.shape
    pred_t = jnp.transpose(predictions, (0, 3, 4, 1, 2))

    anchors = jnp.asarray(_ANCHORS, dtype=jnp.float32)
    cx, cy = bboxes[..., 0], bboxes[..., 1]
    w, h = bboxes[..., 2], bboxes[..., 3]
    valid = ~jnp.all(bboxes == 0.0, axis=-1)
    gx = jnp.clip(jnp.floor(cx * W).astype(jnp.int32), 0, W - 1)
    gy = jnp.clip(jnp.floor(cy * H).astype(jnp.int32), 0, H - 1)
    aw = anchors[:, 0][None, None, :]
    ah = anchors[:, 1][None, None, :]
    inter = jnp.minimum(w[..., None], aw) * jnp.minimum(h[..., None], ah)
    union = (w * h)[..., None] + aw * ah - inter
    iou = inter / (union + 1e-16)
    best = jnp.argmax(iou, axis=-1).astype(jnp.int32)

    pk = (((gy * 64 + gx) * 16 + best) * 2 + valid.astype(jnp.int32)).reshape(
        B, _N, 1
    )

    grid_spec = pltpu.PrefetchScalarGridSpec(
        num_scalar_prefetch=1,
        grid=(B // _NB,),
        in_specs=[
            pl.BlockSpec((_NB, A, 5, H, W), lambda b, s: (b, 0, 0, 0, 0)),
            pl.BlockSpec((B, _N, 4), lambda b, s: (0, 0, 0)),
            pl.BlockSpec((B, _N, 1), lambda b, s: (0, 0, 0)),
        ],
        out_specs=pl.BlockSpec((1, 128), lambda b, s: (0, 0)),
        scratch_shapes=[pltpu.VMEM((_N, 5, _W), jnp.float32)],
    )
    acc = pl.pallas_call(
        _loss_kernel,
        grid_spec=grid_spec,
        out_shape=jax.ShapeDtypeStruct((1, 128), jnp.float32),
    )(pk, pred_t, bboxes, pk)

    cls_loss = jnp.asarray(0.0, jnp.float32)
    return (acc[0, 0], acc[0, 1], acc[0, 2], acc[0, 3], cls_loss)


# speedup vs baseline: 1.0714x; 1.0157x over previous
import math

import jax
import jax.numpy as jnp
from jax.experimental import pallas as pl
from jax.experimental.pallas import tpu as pltpu

_B, _H, _W, _A, _C = 32, 56, 56, 9, 6
_N = 20
_NB = 8
_ANCHORS = [
    (s * math.sqrt(r) / 224.0, s / math.sqrt(r) / 224.0)
    for s in (32, 64, 128)
    for r in (0.5, 1.0, 2.0)
]


def _loss_kernel(idx_smem, pred_ref, bbc_ref, ixc_ref, out_ref, scr):
    g = pl.program_id(0)
    f32 = jnp.float32

    x = pred_ref[:, :, 4, :, :]
    p_all = jax.nn.sigmoid(x)
    s0 = jnp.sum(-jnp.maximum(jnp.log(1.0 - p_all), -100.0))

    npos_b = jnp.float32(0.0)
    sobj_b = jnp.float32(0.0)
    sncorr_b = jnp.float32(0.0)
    scoord_b = jnp.float32(0.0)
    n_c = jax.lax.broadcasted_iota(jnp.int32, (_N, 1), 0)
    n_r = jax.lax.broadcasted_iota(jnp.int32, (1, _N), 1)
    lane = jax.lax.broadcasted_iota(jnp.int32, (_N, _W), 1)

    for i in range(_NB):
        b = g * _NB + i
        bbc = bbc_ref[b]
        cx, cy = bbc[:, 0:1], bbc[:, 1:2]
        w, h = bbc[:, 2:3], bbc[:, 3:4]
        pk = ixc_ref[b]
        valid_c = (pk & 1) != 0
        best_c = (pk >> 1) & 15
        gx_c = (pk >> 5) & 63
        gy_c = (pk >> 11) & 63
        tx = cx * _W - gx_c.astype(f32)
        ty = cy * _H - gy_c.astype(f32)
        baw = jnp.zeros(best_c.shape, f32)
        bah = jnp.zeros(best_c.shape, f32)
        for k, (awk, ahk) in enumerate(_ANCHORS):
            baw = jnp.where(best_c == k, awk, baw)
            bah = jnp.where(best_c == k, ahk, bah)
        tw = jnp.log(w / baw + 1e-16)
        th = jnp.log(h / bah + 1e-16)
        flat_c = (gy_c * _W + gx_c) * _A + best_c

        id_c = jnp.where(valid_c, flat_c, -1 - n_c)
        id_r = jnp.transpose(id_c, (1, 0))
        killed = jnp.any((id_c == id_r) & (n_r > n_c), axis=1, keepdims=True)
        alive = (valid_c & ~killed).astype(f32)

        for n in range(_N):
            s = idx_smem[b, n, 0]
            bestn = (s >> 1) & 15
            gyn = (s >> 11) & 63
            scr[n] = pred_ref[i, bestn, 0:5, gyn, :]
        sel = lane == gx_c
        v = [
            jnp.sum(jnp.where(sel, scr[:, c, :], 0.0), axis=1, keepdims=True)
            for c in range(5)
        ]
        coord_n = (
            (v[0] - tx) ** 2 + (v[1] - ty) ** 2 + (v[2] - tw) ** 2 + (v[3] - th) ** 2
        )
        pv = jax.nn.sigmoid(v[4])
        obj_n = -jnp.maximum(jnp.log(pv), -100.0)
        ncorr_n = -jnp.maximum(jnp.log(1.0 - pv), -100.0)

        npos_b += jnp.sum(alive)
        sobj_b += jnp.sum(alive * obj_n)
        sncorr_b += jnp.sum(alive * ncorr_n)
        scoord_b += jnp.sum(alive * coord_n)

    lane_o = jax.lax.broadcasted_iota(jnp.int32, (1, 128), 1)
    delta = (
        jnp.where(lane_o == 0, s0, 0.0)
        + jnp.where(lane_o == 1, npos_b, 0.0)
        + jnp.where(lane_o == 2, sobj_b, 0.0)
        + jnp.where(lane_o == 3, sncorr_b, 0.0)
        + jnp.where(lane_o == 4, scoord_b, 0.0)
    )

    @pl.when(g == 0)
    def _zero():
        out_ref[...] = jnp.zeros_like(out_ref)

    out_ref[...] = out_ref[...] + delta

    @pl.when(g == _B // _NB - 1)
    def _finalize():
        a = out_ref[...]
        s0s = jnp.sum(jnp.where(lane_o == 0, a, 0.0))
        npos = jnp.sum(jnp.where(lane_o == 1, a, 0.0))
        sobj = jnp.sum(jnp.where(lane_o == 2, a, 0.0))
        sncorr = jnp.sum(jnp.where(lane_o == 3, a, 0.0))
        scoord = jnp.sum(jnp.where(lane_o == 4, a, 0.0))
        n_neg = jnp.float32(_B * _H * _W * _A) - npos
        coord_loss = 5.0 * scoord
        noobj_loss = 0.5 * (s0s - sncorr)
        coord_loss = jnp.where(npos > 0, coord_loss / npos, coord_loss)
        obj_loss = jnp.where(npos > 0, sobj / npos, sobj)
        noobj_loss = jnp.where(n_neg > 0, noobj_loss / n_neg, noobj_loss)
        total_loss = coord_loss + obj_loss + noobj_loss
        out_ref[...] = (
            jnp.where(lane_o == 0, total_loss, 0.0)
            + jnp.where(lane_o == 1, coord_loss, 0.0)
            + jnp.where(lane_o == 2, obj_loss, 0.0)
            + jnp.where(lane_o == 3, noobj_loss, 0.0)
        )


def kernel(predictions, bboxes):
    B, H, W, A, C = predictions.shape
    pred_t = jnp.transpose(predictions, (0, 3, 4, 1, 2))

    anchors = jnp.asarray(_ANCHORS, dtype=jnp.float32)
    cx, cy = bboxes[..., 0], bboxes[..., 1]
    w, h = bboxes[..., 2], bboxes[..., 3]
    valid = ~jnp.all(bboxes == 0.0, axis=-1)
    gx = jnp.clip(jnp.floor(cx * W).astype(jnp.int32), 0, W - 1)
    gy = jnp.clip(jnp.floor(cy * H).astype(jnp.int32), 0, H - 1)
    aw = anchors[:, 0][None, None, :]
    ah = anchors[:, 1][None, None, :]
    inter = jnp.minimum(w[..., None], aw) * jnp.minimum(h[..., None], ah)
    union = (w * h)[..., None] + aw * ah - inter
    iou = inter / (union + 1e-16)
    best = jnp.argmax(iou, axis=-1).astype(jnp.int32)

    pk = (((gy * 64 + gx) * 16 + best) * 2 + valid.astype(jnp.int32)).reshape(
        B, _N, 1
    )

    grid_spec = pltpu.PrefetchScalarGridSpec(
        num_scalar_prefetch=1,
        grid=(B // _NB,),
        in_specs=[
            pl.BlockSpec((_NB, A, 5, H, W), lambda b, s: (b, 0, 0, 0, 0)),
            pl.BlockSpec((B, _N, 4), lambda b, s: (0, 0, 0)),
            pl.BlockSpec((B, _N, 1), lambda b, s: (0, 0, 0)),
        ],
        out_specs=pl.BlockSpec((1, 128), lambda b, s: (0, 0)),
        scratch_shapes=[pltpu.VMEM((_N, 5, _W), jnp.float32)],
    )
    acc = pl.pallas_call(
        _loss_kernel,
        grid_spec=grid_spec,
        out_shape=jax.ShapeDtypeStruct((1, 128), jnp.float32),
    )(pk, pred_t, bboxes, pk)

    cls_loss = jnp.asarray(0.0, jnp.float32)
    return (acc[0, 0], acc[0, 1], acc[0, 2], acc[0, 3], cls_loss)
